# baseline (speedup 1.0000x reference)
import functools

import jax
import jax.numpy as jnp
from jax import lax
from jax.experimental import pallas as pl
from jax.experimental.pallas import tpu as pltpu
from jax.experimental.pallas import tpu_sc as plsc

H, W = 1024, 2048
CROP_H, CROP_W = 720, 1280
SAMPLES = 4
LABEL_COUNT = 19
NC, NS = 2, 16
NWORK = NC * NS
RPW = 23
WB = 1296
NVEC = WB // 16
NBANK = 4
HIST_W = LABEL_COUNT * 16 * NBANK

_mesh = plsc.VectorSubcoreMesh(core_axis_name="c", subcore_axis_name="s")
_params = pltpu.CompilerParams(
    use_tc_tiling_on_sc=False, needs_layout_passes=False)


_TOPS = (219, 196, 73, 29)
_LEFTS = (192, 367, 42, 696)


def _pick(vec, iota, k):
    return jnp.max(jnp.where(iota == k, vec, 0))


@functools.partial(
    pl.kernel,
    out_type=jax.ShapeDtypeStruct((NWORK, SAMPLES * HIST_W), jnp.int32),
    mesh=_mesh,
    scratch_types=[
        pltpu.VMEM((RPW, WB), jnp.int32),
        pltpu.VMEM((SAMPLES * HIST_W,), jnp.int32),
    ],
    compiler_params=_params,
)
def _hist_kernel(label_hbm, out_hbm, buf_v, hist_v):
    w = lax.axis_index("s") * NC + lax.axis_index("c")
    iota = lax.iota(jnp.int32, 16)
    zeros = jnp.zeros((16,), jnp.int32)
    ones = jnp.ones((16,), jnp.int32)

    for k in range(SAMPLES * HIST_W // 16):
        hist_v[pl.ds(k * 16, 16)] = zeros

    lo = jnp.minimum(RPW * w, CROP_H - RPW)
    r_begin = RPW * w

    for c in range(SAMPLES):
        top, left = _TOPS[c], _LEFTS[c]
        left_al = min(left & -16, W - WB)
        shift = left - left_al
        mask_first = iota >= shift
        mask_last = iota < shift

        pltpu.sync_copy(
            label_hbm.at[pl.ds(top + lo, RPW), left_al:left_al + WB], buf_v
        )

        def body(i, carry, _c=c, _mf=mask_first, _ml=mask_last):
            rg = lo + i
            valid = jnp.logical_and(rg >= r_begin, rg < CROP_H)
            rmask = jnp.full((16,), valid)
            m_first = jnp.logical_and(rmask, _mf)
            m_last = jnp.logical_and(rmask, _ml)
            for g in range(0, NVEC, 8):
                js = range(g, min(g + 8, NVEC))
                idxs = []
                for j in js:
                    lv = buf_v[i, pl.ds(j * 16, 16)]
                    base = _c * HIST_W + (j % NBANK) * 16 + iota
                    idxs.append(lv * (16 * NBANK) + base)
                for k, j in enumerate(js):
                    m = m_first if j == 0 else (
                        m_last if j == NVEC - 1 else rmask)
                    plsc.addupdate_scatter(hist_v, [idxs[k]], ones, mask=m)
            return carry

        lax.fori_loop(0, RPW, body, 0)

    pltpu.sync_copy(hist_v, out_hbm.at[w])


_N_BLK = CROP_H // 8


def _tc_crop_body(sel_ref, imga_ref, imgb_ref, laba_ref, labb_ref,
                  oimg_ref, olab_ref):
    rs = sel_ref[1]
    left = sel_ref[2]
    wimg = jnp.concatenate([imga_ref[...], imgb_ref[...]], axis=1)
    wlab = jnp.concatenate([laba_ref[...], labb_ref[...]], axis=0)
    wimg = pltpu.roll(wimg, 16 - rs, 1)[:, :8, :]
    wlab = pltpu.roll(wlab, 16 - rs, 0)[:8, :]
    oimg_ref[...] = pltpu.roll(wimg, W - left, 2)[:, :, :CROP_W]
    olab_ref[...] = pltpu.roll(wlab, W - left, 1)[:, :CROP_W]


_HB = H // 8 - 1

_tc_crop = pl.pallas_call(
    _tc_crop_body,
    grid_spec=pltpu.PrefetchScalarGridSpec(
        num_scalar_prefetch=1,
        grid=(_N_BLK,),
        in_specs=[
            pl.BlockSpec(
                (3, 8, W),
                lambda i, sel: (0, jnp.minimum(sel[0] + i, _HB), 0)),
            pl.BlockSpec(
                (3, 8, W),
                lambda i, sel: (0, jnp.minimum(sel[0] + i + 1, _HB), 0)),
            pl.BlockSpec(
                (8, W), lambda i, sel: (jnp.minimum(sel[0] + i, _HB), 0)),
            pl.BlockSpec(
                (8, W), lambda i, sel: (jnp.minimum(sel[0] + i + 1, _HB), 0)),
        ],
        out_specs=[
            pl.BlockSpec((3, 8, CROP_W), lambda i, sel: (0, i, 0)),
            pl.BlockSpec((8, CROP_W), lambda i, sel: (i, 0)),
        ],
    ),
    out_shape=(
        jax.ShapeDtypeStruct((3, CROP_H, CROP_W), jnp.float32),
        jax.ShapeDtypeStruct((CROP_H, CROP_W), jnp.int32),
    ),
    compiler_params=pltpu.CompilerParams(
        dimension_semantics=("arbitrary",)),
)


def kernel(image, label_image, label_costs):
    label2d = label_image.reshape(H, W)

    parts = _hist_kernel(label2d)
    counts = parts.reshape(
        NWORK, SAMPLES, LABEL_COUNT, NBANK * 16).sum(axis=(0, 3))

    norm_costs = label_costs / jnp.maximum(jnp.sum(jnp.abs(label_costs)), 1e-12)
    total = float(CROP_H * CROP_W)

    def cost_of(c):
        dist = counts[c].astype(jnp.float32) / total
        return jnp.sum(norm_costs * dist)

    best_cost = cost_of(0)
    best_idx = jnp.int32(0)
    for c in range(1, SAMPLES):
        cc = cost_of(c)
        better = cc < best_cost
        best_idx = jnp.where(better, jnp.int32(c), best_idx)
        best_cost = jnp.where(better, cc, best_cost)

    tops_a = jnp.asarray(_TOPS, jnp.int32)
    lefts_a = jnp.asarray(_LEFTS, jnp.int32)
    top = tops_a[best_idx]
    left = lefts_a[best_idx]
    sel = jnp.stack([top // 8, top % 8, left])
    best_image, best_label = _tc_crop(sel, image, image, label2d, label2d)
    return best_image, best_label.reshape(1, CROP_H, CROP_W), best_cost


# speedup vs baseline: 1.3527x; 1.3527x over previous
import functools

import jax
import jax.numpy as jnp
from jax import lax
from jax.experimental import pallas as pl
from jax.experimental.pallas import tpu as pltpu
from jax.experimental.pallas import tpu_sc as plsc

H, W = 1024, 2048
CROP_H, CROP_W = 720, 1280
SAMPLES = 4
LABEL_COUNT = 19
NC, NS = 2, 16
NWORK = NC * NS
ACT = 30
RPW = CROP_H // ACT
SR = 32
WB = 1408
NVEC = WB // 16
NBANK = 4
CROP_PAD = 1280
HR = SAMPLES * CROP_PAD // 128
ROW_CLAMP = H - SR

_TOPS = (219, 196, 73, 29)
_LEFTS = (192, 367, 42, 696)

_mesh = plsc.VectorSubcoreMesh(core_axis_name="c", subcore_axis_name="s")
_params = pltpu.CompilerParams(
    use_tc_tiling_on_sc=True, needs_layout_passes=False)


def _pick(vec, iota, k):
    return jnp.max(jnp.where(iota == k, vec, 0))


def _col_masks(shift, iota_np):
    import numpy as np
    masks = []
    for j in range(NVEC):
        cols = iota_np + 16 * j
        m = (cols >= shift) & (cols < shift + CROP_W)
        masks.append(m)
    return masks


@functools.partial(
    pl.kernel,
    out_type=jax.ShapeDtypeStruct((ACT * HR, 128), jnp.int32),
    mesh=_mesh,
    scratch_types=[
        pltpu.VMEM((SR, WB), jnp.int32),
        pltpu.VMEM((HR, 128), jnp.int32),
    ],
    compiler_params=_params,
)
def _hist_kernel(label_hbm, out_hbm, buf_v, hist_v):
    import numpy as np
    w = lax.axis_index("s") * NC + lax.axis_index("c")
    iota = lax.iota(jnp.int32, 16)
    iota_np = np.arange(16)
    zeros = jnp.zeros((16,), jnp.int32)
    ones = jnp.ones((16,), jnp.int32)

    @pl.when(w < ACT)
    def _():
        for r in range(HR):
            for k in range(128 // 16):
                hist_v[r, pl.ds(k * 16, 16)] = zeros

        for c in range(SAMPLES):
            top, left = _TOPS[c], _LEFTS[c]
            tb8 = top & -8
            rs = top & 7
            lb = min(left & -128, W - WB)
            shift = left - lb
            masks_np = _col_masks(shift, iota_np)
            js = [j for j in range(NVEC) if masks_np[j].any()]

            base8 = tb8 + RPW * w
            start = pl.multiple_of(jnp.minimum(base8, ROW_CLAMP), 8)
            roff = base8 - start + rs

            pltpu.sync_copy(
                label_hbm.at[pl.ds(start, SR), lb:lb + WB], buf_v)

            def body(i, carry, _c=c, _js=js, _masks=masks_np, _roff=roff):
                r = _roff + i
                for g in range(0, len(_js), 8):
                    grp = _js[g:g + 8]
                    rows, cols, ms = [], [], []
                    for j in grp:
                        lv = buf_v[r, pl.ds(j * 16, 16)]
                        bank = (j % NBANK) * 16
                        idx = (lv * (16 * NBANK)
                               + (_c * CROP_PAD + bank + iota))
                        rows.append(idx >> 7)
                        cols.append(idx & 127)
                        if _masks[j].all():
                            ms.append(None)
                        else:
                            cols16 = iota + (16 * j)
                            ms.append(jnp.logical_and(
                                cols16 >= shift, cols16 < shift + CROP_W))
                    for k in range(len(grp)):
                        plsc.addupdate_scatter(
                            hist_v, [rows[k], cols[k]], ones, mask=ms[k])
                return carry

            lax.fori_loop(0, RPW, body, 0)

        pltpu.sync_copy(
            hist_v, out_hbm.at[pl.ds(pl.multiple_of(HR * w, 8), HR), :])


@functools.partial(
    pl.kernel,
    out_type=(
        jax.ShapeDtypeStruct((3, CROP_H, CROP_W), jnp.int32),
        jax.ShapeDtypeStruct((CROP_H, CROP_W), jnp.int32),
    ),
    mesh=_mesh,
    scratch_types=[
        pltpu.VMEM((16,), jnp.int32),
        pltpu.VMEM((SR, WB), jnp.int32),
        pltpu.VMEM((RPW, CROP_W), jnp.int32),
    ],
    compiler_params=_params,
)
def _crop_kernel(img_hbm, lab_hbm, sel_hbm, oimg_hbm, olab_hbm,
                 sel_v, buf_v, obuf_v):
    w = lax.axis_index("s") * NC + lax.axis_index("c")
    iota = lax.iota(jnp.int32, 16)

    @pl.when(w < ACT)
    def _():
        pltpu.sync_copy(sel_hbm, sel_v)
        sv = sel_v[...]
        top = _pick(sv, iota, 0)
        left = _pick(sv, iota, 1)
        rs = top & 7
        lb = pl.multiple_of(jnp.minimum(left & -128, W - WB), 128)
        shift = left - lb
        base8 = (top & -8) + RPW * w
        start = pl.multiple_of(jnp.minimum(base8, ROW_CLAMP), 8)
        roff = base8 - start + rs
        olo = pl.multiple_of(RPW * w, 8)
        cbase = shift + iota

        def shift_rows():
            def body(i, carry):
                rowv = jnp.full((16,), roff + i)
                for g in range(0, CROP_W // 16, 8):
                    vs = [plsc.load_gather(
                        buf_v, [rowv, cbase + ((g + k) * 16)])
                        for k in range(8)]
                    for k in range(8):
                        obuf_v[i, pl.ds((g + k) * 16, 16)] = vs[k]
                return carry
            lax.fori_loop(0, RPW, body, 0)

        for ch in range(3):
            pltpu.sync_copy(
                img_hbm.at[ch, pl.ds(start, SR), pl.ds(lb, WB)], buf_v)
            shift_rows()
            pltpu.sync_copy(obuf_v, oimg_hbm.at[ch, pl.ds(olo, RPW), :])

        pltpu.sync_copy(
            lab_hbm.at[pl.ds(start, SR), pl.ds(lb, WB)], buf_v)
        shift_rows()
        pltpu.sync_copy(obuf_v, olab_hbm.at[pl.ds(olo, RPW), :])


def kernel(image, label_image, label_costs):
    label2d = label_image.reshape(H, W)
    image_i = jax.lax.bitcast_convert_type(image, jnp.int32)

    parts = _hist_kernel(label2d)
    counts = parts.reshape(ACT, SAMPLES, CROP_PAD)[:, :, :LABEL_COUNT * 64]
    counts = counts.reshape(ACT, SAMPLES, LABEL_COUNT, 64).sum(axis=(0, 3))

    norm_costs = label_costs / jnp.maximum(jnp.sum(jnp.abs(label_costs)), 1e-12)
    total = float(CROP_H * CROP_W)

    def cost_of(c):
        dist = counts[c].astype(jnp.float32) / total
        return jnp.sum(norm_costs * dist)

    best_cost = cost_of(0)
    best_idx = jnp.int32(0)
    for c in range(1, SAMPLES):
        cc = cost_of(c)
        better = cc < best_cost
        best_idx = jnp.where(better, jnp.int32(c), best_idx)
        best_cost = jnp.where(better, cc, best_cost)

    tops_a = jnp.asarray(_TOPS, jnp.int32)
    lefts_a = jnp.asarray(_LEFTS, jnp.int32)
    sel = jnp.zeros((16,), jnp.int32)
    sel = sel.at[0].set(tops_a[best_idx]).at[1].set(lefts_a[best_idx])

    best_image_i, best_label = _crop_kernel(image_i, label2d, sel)
    best_image = jax.lax.bitcast_convert_type(best_image_i, jnp.float32)
    return best_image, best_label.reshape(1, CROP_H, CROP_W), best_cost


# speedup vs baseline: 1.6309x; 1.2057x over previous
import functools

import jax
import jax.numpy as jnp
from jax import lax
from jax.experimental import pallas as pl
from jax.experimental.pallas import tpu as pltpu
from jax.experimental.pallas import tpu_sc as plsc

H, W = 1024, 2048
CROP_H, CROP_W = 720, 1280
SAMPLES = 4
LABEL_COUNT = 19
NC, NS = 2, 16
NWORK = NC * NS
ACT = 30
RPW = CROP_H // ACT
SR = 32
WB = 1408
NVEC = WB // 16
NBANK = 4
CROP_PAD = 1280
HR = SAMPLES * CROP_PAD // 128
ROW_CLAMP = H - SR

_TOPS = (219, 196, 73, 29)
_LEFTS = (192, 367, 42, 696)

_mesh = plsc.VectorSubcoreMesh(core_axis_name="c", subcore_axis_name="s")
_params = pltpu.CompilerParams(
    use_tc_tiling_on_sc=True, needs_layout_passes=False)


def _pick(vec, iota, k):
    return jnp.max(jnp.where(iota == k, vec, 0))


def _col_masks(shift, iota_np):
    import numpy as np
    masks = []
    for j in range(NVEC):
        cols = iota_np + 16 * j
        m = (cols >= shift) & (cols < shift + CROP_W)
        masks.append(m)
    return masks


@functools.partial(
    pl.kernel,
    out_type=jax.ShapeDtypeStruct((ACT * SAMPLES * CROP_PAD,), jnp.int32),
    mesh=_mesh,
    scratch_types=[
        pltpu.VMEM((SR, WB), jnp.int32),
        pltpu.VMEM((SR, WB), jnp.int32),
        pltpu.VMEM((SAMPLES * CROP_PAD,), jnp.int32),
        pltpu.SemaphoreType.DMA,
        pltpu.SemaphoreType.DMA,
    ],
    compiler_params=_params,
)
def _hist_kernel(label_hbm, out_hbm, buf0_v, buf1_v, hist_v, sem0, sem1):
    import numpy as np
    w = lax.axis_index("s") * NC + lax.axis_index("c")
    iota = lax.iota(jnp.int32, 16)
    iota_np = np.arange(16)
    zeros = jnp.zeros((16,), jnp.int32)
    ones = jnp.ones((16,), jnp.int32)
    bufs = (buf0_v, buf1_v)
    sems = (sem0, sem1)

    @pl.when(w < ACT)
    def _():
        for k in range(SAMPLES * CROP_PAD // 16):
            hist_v[pl.ds(k * 16, 16)] = zeros

        def src_roff(c):
            top, left = _TOPS[c], _LEFTS[c]
            lb = min(left & -128, W - WB)
            base8 = (top & -8) + RPW * w
            start = pl.multiple_of(jnp.minimum(base8, ROW_CLAMP), 8)
            roff = base8 - start + (top & 7)
            return label_hbm.at[pl.ds(start, SR), lb:lb + WB], roff

        src0, roff0 = src_roff(0)
        pend = pltpu.async_copy(src0, bufs[0], sems[0])
        roffs = [roff0]
        for c in range(SAMPLES):
            if c + 1 < SAMPLES:
                srcn, roffn = src_roff(c + 1)
                nxt = pltpu.async_copy(srcn, bufs[(c + 1) % 2], sems[(c + 1) % 2])
                roffs.append(roffn)
            pend.wait()
            if c + 1 < SAMPLES:
                pend = nxt

            top, left = _TOPS[c], _LEFTS[c]
            lb = min(left & -128, W - WB)
            shift = left - lb
            masks_np = _col_masks(shift, iota_np)
            js = [j for j in range(NVEC) if masks_np[j].any()]
            buf_v = bufs[c % 2]
            roff = roffs[c]

            def body(i, carry, _c=c, _js=js, _masks=masks_np,
                     _roff=roff, _buf=buf_v, _shift=shift):
                r = _roff + i
                for g in range(0, len(_js), 8):
                    grp = _js[g:g + 8]
                    idxs, ms = [], []
                    for j in grp:
                        lv = _buf[r, pl.ds(j * 16, 16)]
                        bank = (j % NBANK) * 16
                        idxs.append(lv * (16 * NBANK)
                                    + (_c * CROP_PAD + bank + iota))
                        if _masks[j].all():
                            ms.append(None)
                        else:
                            cols16 = iota + (16 * j)
                            ms.append(jnp.logical_and(
                                cols16 >= _shift, cols16 < _shift + CROP_W))
                    for k in range(len(grp)):
                        plsc.addupdate_scatter(
                            hist_v, [idxs[k]], ones, mask=ms[k])
                return carry

            lax.fori_loop(0, RPW, body, 0)

        pltpu.sync_copy(
            hist_v,
            out_hbm.at[pl.ds(
                pl.multiple_of(SAMPLES * CROP_PAD * w, 8),
                SAMPLES * CROP_PAD)])


@functools.partial(
    pl.kernel,
    out_type=(
        jax.ShapeDtypeStruct((3, CROP_H, CROP_W), jnp.float32),
        jax.ShapeDtypeStruct((CROP_H, CROP_W), jnp.int32),
    ),
    mesh=_mesh,
    scratch_types=[
        pltpu.VMEM((16,), jnp.int32),
        pltpu.VMEM((SR, WB), jnp.float32),
        pltpu.VMEM((RPW, CROP_W), jnp.float32),
        pltpu.VMEM((RPW, CROP_W), jnp.int32),
    ],
    compiler_params=_params,
)
def _crop_kernel(img_hbm, labf_hbm, sel_hbm, oimg_hbm, olab_hbm,
                 sel_v, buf_v, obuf_v, olbuf_v):
    w = lax.axis_index("s") * NC + lax.axis_index("c")
    iota = lax.iota(jnp.int32, 16)

    @pl.when(w < ACT)
    def _():
        pltpu.sync_copy(sel_hbm, sel_v)
        sv = sel_v[...]
        top = _pick(sv, iota, 0)
        left = _pick(sv, iota, 1)
        rs = top & 7
        lb = pl.multiple_of(jnp.minimum(left & -128, W - WB), 128)
        shift = left - lb
        base8 = (top & -8) + RPW * w
        start = pl.multiple_of(jnp.minimum(base8, ROW_CLAMP), 8)
        roff = base8 - start + rs
        olo = pl.multiple_of(RPW * w, 8)
        cbase = shift + iota

        def shift_rows(as_label):
            def body(i, carry):
                rowv = jnp.full((16,), roff + i)
                for g in range(0, CROP_W // 16, 8):
                    vs = [plsc.load_gather(
                        buf_v, [rowv, cbase + ((g + k) * 16)])
                        for k in range(8)]
                    for k in range(8):
                        if as_label:
                            olbuf_v[i, pl.ds((g + k) * 16, 16)] = (
                                plsc.bitcast(vs[k], jnp.int32))
                        else:
                            obuf_v[i, pl.ds((g + k) * 16, 16)] = vs[k]
                return carry
            lax.fori_loop(0, RPW, body, 0)

        for ch in range(3):
            pltpu.sync_copy(
                img_hbm.at[ch, pl.ds(start, SR), pl.ds(lb, WB)], buf_v)
            shift_rows(False)
            pltpu.sync_copy(obuf_v, oimg_hbm.at[ch, pl.ds(olo, RPW), :])

        pltpu.sync_copy(
            labf_hbm.at[pl.ds(start, SR), pl.ds(lb, WB)], buf_v)
        shift_rows(True)
        pltpu.sync_copy(olbuf_v, olab_hbm.at[pl.ds(olo, RPW), :])


def kernel(image, label_image, label_costs):
    label2d = label_image.reshape(H, W)
    label_f = jax.lax.bitcast_convert_type(label2d, jnp.float32)

    parts = _hist_kernel(label2d)
    counts = parts.reshape(ACT, SAMPLES, CROP_PAD)[:, :, :LABEL_COUNT * 64]
    counts = counts.reshape(ACT, SAMPLES, LABEL_COUNT, 64).sum(axis=(0, 3))

    norm_costs = label_costs / jnp.maximum(jnp.sum(jnp.abs(label_costs)), 1e-12)
    total = float(CROP_H * CROP_W)

    def cost_of(c):
        dist = counts[c].astype(jnp.float32) / total
        return jnp.sum(norm_costs * dist)

    best_cost = cost_of(0)
    best_idx = jnp.int32(0)
    for c in range(1, SAMPLES):
        cc = cost_of(c)
        better = cc < best_cost
        best_idx = jnp.where(better, jnp.int32(c), best_idx)
        best_cost = jnp.where(better, cc, best_cost)

    tops_a = jnp.asarray(_TOPS, jnp.int32)
    lefts_a = jnp.asarray(_LEFTS, jnp.int32)
    sel = jnp.zeros((16,), jnp.int32)
    sel = sel.at[0].set(tops_a[best_idx]).at[1].set(lefts_a[best_idx])

    best_image, best_label = _crop_kernel(image, label_f, sel)
    return best_image, best_label.reshape(1, CROP_H, CROP_W), best_cost
